# Initial kernel scaffold; baseline (speedup 1.0000x reference)
import functools

import jax
import jax.numpy as jnp
import numpy as np
from jax import lax
from jax.experimental import pallas as pl
from jax.experimental.pallas import tpu as pltpu
from jax.experimental.pallas import tpu_sc as plsc

NNODES = 10000
NPAD = 10240
DIN = 128
NH = 8
ETOT = 330000
NC, NS = 2, 16
NW = NC * NS
OC = 128
CH = (ETOT + NW * OC - 1) // (NW * OC)
EPAD = NW * CH * OC
RPT = NPAD // NS
BLK = 512


def _lrelu(a):
    return jnp.where(a > 0, a, 0.2 * a)


def _k1_body(x_ref, w1p_ref, m1s_ref, m1d_ref, xl_ref, as_ref, ad_ref):
    x = x_ref[...]
    xl = jnp.dot(x, w1p_ref[...], preferred_element_type=jnp.float32)
    xl_ref[...] = xl
    as_ref[...] = jnp.dot(xl, m1s_ref[...], preferred_element_type=jnp.float32)
    ad_ref[...] = jnp.dot(xl, m1d_ref[...], preferred_element_type=jnp.float32)


def _k3_body(up_ref, dp_ref, s16_ref, b1_ref, w2p_ref, m2s_ref, m2d_ref,
             xl2_ref, as2_ref, ad2_ref):
    u = up_ref[0] + up_ref[1]
    d = dp_ref[0] + dp_ref[1]
    d128 = jnp.dot(d, s16_ref[...], preferred_element_type=jnp.float32)
    hpre = u / (d128 + 1e-16) + b1_ref[...]
    h = jnp.where(hpre > 0, hpre, 0.2 * (jnp.exp(hpre) - 1.0))
    xl2 = jnp.dot(h, w2p_ref[...], preferred_element_type=jnp.float32)
    xl2_ref[...] = xl2
    as2_ref[...] = jnp.dot(xl2, m2s_ref[...], preferred_element_type=jnp.float32)
    ad2_ref[...] = jnp.dot(xl2, m2d_ref[...], preferred_element_type=jnp.float32)


def _k4b_body(dp_ref, r_ref):
    r_ref[...] = 1.0 / (dp_ref[0] + dp_ref[1] + 1e-16)


def _k6_body(op_ref, b2_ref, o_ref):
    o_ref[...] = (op_ref[0] + op_ref[1]) * 0.125 + b2_ref[...]


_MESH = plsc.VectorSubcoreMesh(core_axis_name="c", subcore_axis_name="s")


def _zero_zbuf(zbuf, cols):
    z16 = jnp.zeros((16,), jnp.float32)
    for i in range(16):
        for j in range(cols // 16):
            zbuf[i, pl.ds(16 * j, 16)] = z16


def _zero_shared(zbuf, sh_ref, row0):
    def _z(k, carry):
        pltpu.sync_copy(zbuf, sh_ref.at[pl.ds(row0 + 16 * k, 16)])
        return carry
    lax.fori_loop(0, RPT // 16, _z, 0)


def _writeout(sh_ref, out_ref, core, row0):
    def _w(k, carry):
        r = row0 + 16 * k
        pltpu.sync_copy(sh_ref.at[pl.ds(r, 16)], out_ref.at[core, pl.ds(r, 16)])
        return carry
    lax.fori_loop(0, RPT // 16, _w, 0)


@functools.partial(
    pl.kernel, mesh=_MESH,
    out_type=(jax.ShapeDtypeStruct((NC, NPAD, 128), jnp.float32),
              jax.ShapeDtypeStruct((NC, NPAD, 16), jnp.float32)),
    scratch_types=[
        pltpu.VMEM((OC,), jnp.int32),
        pltpu.VMEM((OC,), jnp.int32),
        pltpu.VMEM((OC, 16), jnp.float32),
        pltpu.VMEM((OC, 16), jnp.float32),
        pltpu.VMEM((OC, 16), jnp.float32),
        pltpu.VMEM((OC, 128), jnp.float32),
        pltpu.VMEM((OC, 128), jnp.float32),
        pltpu.VMEM((16, 128), jnp.float32),
        pltpu.VMEM((16, 16), jnp.float32),
        pltpu.VMEM_SHARED((NPAD, 128), jnp.float32),
        pltpu.VMEM_SHARED((NPAD, 16), jnp.float32),
        pltpu.SemaphoreType.DMA,
    ],
)
def _edge1(src_hbm, dst_hbm, xl1_hbm, as1_hbm, ad1_hbm, u_out, d_out,
           idx_s, idx_d, as_v, ad_v, ex_v, xl_v, msg_v, zbuf, zbuf16,
           u_sh, d_sh, sem):
    c = lax.axis_index("c")
    s = lax.axis_index("s")
    w = c * NS + s
    row0 = s * RPT
    _zero_zbuf(zbuf, 128)
    _zero_zbuf(zbuf16, 16)
    _zero_shared(zbuf, u_sh, row0)
    _zero_shared(zbuf16, d_sh, row0)
    plsc.subcore_barrier()

    def _chunk(k, carry):
        base = (w * CH + k) * OC
        pltpu.sync_copy(src_hbm.at[pl.ds(base, OC)], idx_s)
        pltpu.sync_copy(dst_hbm.at[pl.ds(base, OC)], idx_d)
        pltpu.async_copy(as1_hbm.at[idx_s], as_v, sem).wait()
        pltpu.async_copy(ad1_hbm.at[idx_d], ad_v, sem).wait()
        pltpu.async_copy(xl1_hbm.at[idx_s], xl_v, sem).wait()

        def _edge(b, cin):
            ex = jnp.exp(_lrelu(as_v[b, :] + ad_v[b, :]))
            ex_v[b, :] = ex
            for h in range(NH):
                sc = ex_v[b, h]
                msg_v[b, pl.ds(16 * h, 16)] = xl_v[b, pl.ds(16 * h, 16)] * sc
            return cin
        lax.fori_loop(0, OC, _edge, 0)
        pltpu.sync_copy(msg_v, u_sh.at[idx_d], add=True)
        pltpu.sync_copy(ex_v, d_sh.at[idx_d], add=True)
        return carry
    lax.fori_loop(0, CH, _chunk, 0)
    plsc.subcore_barrier()
    _writeout(u_sh, u_out, c, row0)
    _writeout(d_sh, d_out, c, row0)


@functools.partial(
    pl.kernel, mesh=_MESH,
    out_type=jax.ShapeDtypeStruct((NC, NPAD, 16), jnp.float32),
    scratch_types=[
        pltpu.VMEM((OC,), jnp.int32),
        pltpu.VMEM((OC,), jnp.int32),
        pltpu.VMEM((OC, 16), jnp.float32),
        pltpu.VMEM((OC, 16), jnp.float32),
        pltpu.VMEM((OC, 16), jnp.float32),
        pltpu.VMEM((16, 16), jnp.float32),
        pltpu.VMEM_SHARED((NPAD, 16), jnp.float32),
        pltpu.SemaphoreType.DMA,
    ],
)
def _edge2_denom(src_hbm, dst_hbm, as2_hbm, ad2_hbm, d_out,
                 idx_s, idx_d, as_v, ad_v, ex_v, zbuf16, d_sh, sem):
    c = lax.axis_index("c")
    s = lax.axis_index("s")
    w = c * NS + s
    row0 = s * RPT
    _zero_zbuf(zbuf16, 16)
    _zero_shared(zbuf16, d_sh, row0)
    plsc.subcore_barrier()

    def _chunk(k, carry):
        base = (w * CH + k) * OC
        pltpu.sync_copy(src_hbm.at[pl.ds(base, OC)], idx_s)
        pltpu.sync_copy(dst_hbm.at[pl.ds(base, OC)], idx_d)
        pltpu.async_copy(as2_hbm.at[idx_s], as_v, sem).wait()
        pltpu.async_copy(ad2_hbm.at[idx_d], ad_v, sem).wait()

        def _edge(b, cin):
            ex_v[b, :] = jnp.exp(_lrelu(as_v[b, :] + ad_v[b, :]))
            return cin
        lax.fori_loop(0, OC, _edge, 0)
        pltpu.sync_copy(ex_v, d_sh.at[idx_d], add=True)
        return carry
    lax.fori_loop(0, CH, _chunk, 0)
    plsc.subcore_barrier()
    _writeout(d_sh, d_out, c, row0)


@functools.partial(
    pl.kernel, mesh=_MESH,
    out_type=jax.ShapeDtypeStruct((NC, NPAD, 128), jnp.float32),
    scratch_types=[
        pltpu.VMEM((4, 32), jnp.int32),
        pltpu.VMEM((4, 32), jnp.int32),
        pltpu.VMEM((OC,), jnp.int32),
        pltpu.VMEM((OC,), jnp.int32),
        pltpu.VMEM((OC, 16), jnp.float32),
        pltpu.VMEM((OC, 16), jnp.float32),
        pltpu.VMEM((OC, 16), jnp.float32),
        pltpu.VMEM((OC, 16), jnp.float32),
        pltpu.VMEM((32, 1024), jnp.float32),
        pltpu.VMEM((32, 128), jnp.float32),
        pltpu.VMEM((16, 128), jnp.float32),
        pltpu.VMEM_SHARED((NPAD, 128), jnp.float32),
        pltpu.SemaphoreType.DMA,
    ],
)
def _edge2_msg(src_hbm, dst_hbm, src2d_hbm, dst2d_hbm, xl2_hbm, as2_hbm,
               ad2_hbm, r2_hbm, o_out,
               idx_s2, idx_d2, idx_s, idx_d, as_v, ad_v, r_v, at_v,
               xl_v, msg_v, zbuf, o_sh, sem):
    c = lax.axis_index("c")
    s = lax.axis_index("s")
    w = c * NS + s
    row0 = s * RPT
    _zero_zbuf(zbuf, 128)
    _zero_shared(zbuf, o_sh, row0)
    plsc.subcore_barrier()

    def _chunk(k, carry):
        base = (w * CH + k) * OC
        pltpu.sync_copy(src_hbm.at[pl.ds(base, OC)], idx_s)
        pltpu.sync_copy(dst_hbm.at[pl.ds(base, OC)], idx_d)
        pltpu.sync_copy(src2d_hbm.at[pl.ds(base // 32, 4)], idx_s2)
        pltpu.sync_copy(dst2d_hbm.at[pl.ds(base // 32, 4)], idx_d2)
        pltpu.async_copy(as2_hbm.at[idx_s], as_v, sem).wait()
        pltpu.async_copy(ad2_hbm.at[idx_d], ad_v, sem).wait()
        pltpu.async_copy(r2_hbm.at[idx_d], r_v, sem).wait()

        def _att(b, cin):
            at_v[b, :] = jnp.exp(_lrelu(as_v[b, :] + ad_v[b, :])) * r_v[b, :]
            return cin
        lax.fori_loop(0, OC, _att, 0)

        for sub in range(4):
            pltpu.async_copy(xl2_hbm.at[idx_s2.at[sub]], xl_v, sem).wait()

            def _edge(b, cin):
                bb = 32 * sub + b
                att = [at_v[bb, h] for h in range(NH)]
                for j in range(8):
                    acc = xl_v[b, pl.ds(16 * j, 16)] * att[0]
                    for h in range(1, NH):
                        acc = acc + xl_v[b, pl.ds(128 * h + 16 * j, 16)] * att[h]
                    msg_v[b, pl.ds(16 * j, 16)] = acc
                return cin
            lax.fori_loop(0, 32, _edge, 0)
            pltpu.sync_copy(msg_v, o_sh.at[idx_d2.at[sub]], add=True)
        return carry
    lax.fori_loop(0, CH, _chunk, 0)
    plsc.subcore_barrier()
    _writeout(o_sh, o_out, c, row0)


def _dup16(m):
    eye = jnp.eye(NH, dtype=jnp.float32)
    blk = m[:, :, None] * eye[:, None, :]
    blk = jnp.concatenate([blk, blk], axis=-1)
    blk = jnp.pad(blk, ((0, 0), (0, 8), (0, 0)))
    return blk.reshape(128, 16)


def _dup2(m):
    eye = jnp.eye(NH, dtype=jnp.float32)
    blk = m[:, :, None] * eye[:, None, :]
    blk = jnp.concatenate([blk, blk], axis=-1)
    return blk.reshape(1024, 16)


def kernel(x, edge_index, W1, att_src1, att_dst1, bias1, W2, att_src2,
           att_dst2, bias2):
    f32 = jnp.float32
    cols = np.arange(64).reshape(8, 8)
    cols = (cols // 8 * 16 + cols % 8).reshape(-1)
    w1p = jnp.zeros((DIN, 128), f32).at[:, cols].set(W1)
    w2p = jnp.zeros((128, 1024), f32).at[cols].set(W2)
    b1p = jnp.zeros((128,), f32).at[cols].set(bias1)
    m1s, m1d = _dup16(att_src1), _dup16(att_dst1)
    m2s, m2d = _dup2(att_src2), _dup2(att_dst2)
    s16 = np.zeros((16, 128), np.float32)
    for h in range(NH):
        s16[h, h * 16:h * 16 + 8] = 1.0
    s16 = jnp.asarray(s16)

    xp = jnp.pad(x, ((0, NPAD - NNODES), (0, 0)))
    loop = jnp.arange(NNODES, dtype=edge_index.dtype)
    pad = jnp.full((EPAD - ETOT,), NNODES, dtype=edge_index.dtype)
    src = jnp.concatenate([edge_index[0], loop, pad])
    dst = jnp.concatenate([edge_index[1], loop, pad])
    src2d = src.reshape(EPAD // 32, 32)
    dst2d = dst.reshape(EPAD // 32, 32)

    nblk = NPAD // BLK
    xl1, as1, ad1 = pl.pallas_call(
        _k1_body,
        grid=(nblk,),
        in_specs=[
            pl.BlockSpec((BLK, DIN), lambda i: (i, 0)),
            pl.BlockSpec((DIN, 128), lambda i: (0, 0)),
            pl.BlockSpec((128, 16), lambda i: (0, 0)),
            pl.BlockSpec((128, 16), lambda i: (0, 0)),
        ],
        out_specs=[
            pl.BlockSpec((BLK, 128), lambda i: (i, 0)),
            pl.BlockSpec((BLK, 16), lambda i: (i, 0)),
            pl.BlockSpec((BLK, 16), lambda i: (i, 0)),
        ],
        out_shape=[
            jax.ShapeDtypeStruct((NPAD, 128), f32),
            jax.ShapeDtypeStruct((NPAD, 16), f32),
            jax.ShapeDtypeStruct((NPAD, 16), f32),
        ],
    )(xp, w1p, m1s, m1d)

    u_p, d_p = _edge1(src, dst, xl1, as1, ad1)

    xl2, as2, ad2 = pl.pallas_call(
        _k3_body,
        grid=(nblk,),
        in_specs=[
            pl.BlockSpec((NC, BLK, 128), lambda i: (0, i, 0)),
            pl.BlockSpec((NC, BLK, 16), lambda i: (0, i, 0)),
            pl.BlockSpec((16, 128), lambda i: (0, 0)),
            pl.BlockSpec((128,), lambda i: (0,)),
            pl.BlockSpec((128, 1024), lambda i: (0, 0)),
            pl.BlockSpec((1024, 16), lambda i: (0, 0)),
            pl.BlockSpec((1024, 16), lambda i: (0, 0)),
        ],
        out_specs=[
            pl.BlockSpec((BLK, 1024), lambda i: (i, 0)),
            pl.BlockSpec((BLK, 16), lambda i: (i, 0)),
            pl.BlockSpec((BLK, 16), lambda i: (i, 0)),
        ],
        out_shape=[
            jax.ShapeDtypeStruct((NPAD, 1024), f32),
            jax.ShapeDtypeStruct((NPAD, 16), f32),
            jax.ShapeDtypeStruct((NPAD, 16), f32),
        ],
    )(u_p, d_p, s16, b1p, w2p, m2s, m2d)

    d2_p = _edge2_denom(src, dst, as2, ad2)

    r2 = pl.pallas_call(
        _k4b_body,
        grid=(4,),
        in_specs=[pl.BlockSpec((NC, NPAD // 4, 16), lambda i: (0, i, 0))],
        out_specs=pl.BlockSpec((NPAD // 4, 16), lambda i: (i, 0)),
        out_shape=jax.ShapeDtypeStruct((NPAD, 16), f32),
    )(d2_p)

    o_p = _edge2_msg(src, dst, src2d, dst2d, xl2, as2, ad2, r2)

    out = pl.pallas_call(
        _k6_body,
        grid=(4,),
        in_specs=[
            pl.BlockSpec((NC, NPAD // 4, 128), lambda i: (0, i, 0)),
            pl.BlockSpec((128,), lambda i: (0,)),
        ],
        out_specs=pl.BlockSpec((NPAD // 4, 128), lambda i: (i, 0)),
        out_shape=jax.ShapeDtypeStruct((NPAD, 128), f32),
    )(o_p, bias2)

    return out[:NNODES]


# speedup vs baseline: 23.5405x; 23.5405x over previous
import functools

import jax
import jax.numpy as jnp
import numpy as np
from jax import lax
from jax.experimental import pallas as pl
from jax.experimental.pallas import tpu as pltpu
from jax.experimental.pallas import tpu_sc as plsc

NNODES = 10000
NPAD = 10240
DIN = 128
NH = 8
ETOT = 330000
NC, NS = 2, 16
NW = NC * NS
OC = 128
CH = (ETOT + NW * OC - 1) // (NW * OC)
EPAD = NW * CH * OC
RPT = NPAD // NS
BLK = 512


def _lrelu(a):
    return jnp.where(a > 0, a, 0.2 * a)


def _k1_body(x_ref, w1p_ref, m1s_ref, m1d_ref, ones_ref, xl_ref, as_ref,
             ad_ref):
    x = x_ref[...]
    xl = jnp.dot(x, w1p_ref[...], preferred_element_type=jnp.float32)
    xl_ref[...] = xl + ones_ref[...]
    as_ref[...] = jnp.dot(xl, m1s_ref[...], preferred_element_type=jnp.float32)
    ad_ref[...] = jnp.dot(xl, m1d_ref[...], preferred_element_type=jnp.float32)


def _k3_body(up_ref, s2_ref, b1_ref, w2p_ref, m2s_ref, m2d_ref,
             xl2_ref, as2_ref, ad2_ref):
    u = up_ref[0] + up_ref[1]
    d128 = jnp.dot(u, s2_ref[...], preferred_element_type=jnp.float32)
    hpre = u / (d128 + 1e-16) + b1_ref[...]
    h = jnp.where(hpre > 0, hpre, 0.2 * (jnp.exp(hpre) - 1.0))
    xl2 = jnp.dot(h, w2p_ref[...], preferred_element_type=jnp.float32)
    xl2_ref[...] = xl2
    as2_ref[...] = jnp.dot(xl2, m2s_ref[...], preferred_element_type=jnp.float32)
    ad2_ref[...] = jnp.dot(xl2, m2d_ref[...], preferred_element_type=jnp.float32)


def _k4b_body(dp_ref, r_ref):
    r_ref[...] = 1.0 / (dp_ref[0] + dp_ref[1] + 1e-16)


def _k6_body(op_ref, b2_ref, o_ref):
    o_ref[...] = (op_ref[0] + op_ref[1]) * 0.125 + b2_ref[...]


_MESH = plsc.VectorSubcoreMesh(core_axis_name="c", subcore_axis_name="s")


def _zero_zbuf(zbuf, cols):
    z16 = jnp.zeros((16,), jnp.float32)
    for i in range(16):
        for j in range(cols // 16):
            zbuf[i, pl.ds(16 * j, 16)] = z16


def _zero_shared(zbuf, sh_ref, row0):
    def _z(k, carry):
        pltpu.sync_copy(zbuf, sh_ref.at[pl.ds(row0 + 16 * k, 16)])
        return carry
    lax.fori_loop(0, RPT // 16, _z, 0)


def _writeout(sh_ref, out_ref, core, row0):
    def _w(k, carry):
        r = row0 + 16 * k
        pltpu.sync_copy(sh_ref.at[pl.ds(r, 16)], out_ref.at[core, pl.ds(r, 16)])
        return carry
    lax.fori_loop(0, RPT // 16, _w, 0)


@functools.partial(
    pl.kernel, mesh=_MESH,
    compiler_params=pltpu.CompilerParams(use_tc_tiling_on_sc=False),
    out_type=jax.ShapeDtypeStruct((NC, NPAD, 128), jnp.float32),
    scratch_types=[
        pltpu.VMEM((OC,), jnp.int32),
        pltpu.VMEM((OC,), jnp.int32),
        pltpu.VMEM((OC, 16), jnp.float32),
        pltpu.VMEM((OC, 16), jnp.float32),
        pltpu.VMEM((OC, 128), jnp.float32),
        pltpu.VMEM((OC, 128), jnp.float32),
        pltpu.VMEM((16, 128), jnp.float32),
        pltpu.VMEM_SHARED((NPAD, 128), jnp.float32),
        pltpu.SemaphoreType.DMA,
    ],
)
def _edge1(src_hbm, dst_hbm, xl1_hbm, as1_hbm, ad1_hbm, u_out,
           idx_s, idx_d, as_v, ad_v, xl_v, msg_v, zbuf, u_sh, sem):
    c = lax.axis_index("c")
    s = lax.axis_index("s")
    w = c * NS + s
    row0 = s * RPT
    _zero_zbuf(zbuf, 128)
    _zero_shared(zbuf, u_sh, row0)
    plsc.subcore_barrier()

    def _chunk(k, carry):
        base = (w * CH + k) * OC
        pltpu.sync_copy(src_hbm.at[pl.ds(base, OC)], idx_s)
        pltpu.sync_copy(dst_hbm.at[pl.ds(base, OC)], idx_d)
        pltpu.async_copy(as1_hbm.at[idx_s], as_v, sem).wait()
        pltpu.async_copy(ad1_hbm.at[idx_d], ad_v, sem).wait()
        pltpu.async_copy(xl1_hbm.at[idx_s], xl_v, sem).wait()

        def _edge(b, cin):
            ex = jnp.exp(_lrelu(as_v[b, :] + ad_v[b, :]))
            for h in range(NH):
                msg_v[b, pl.ds(16 * h, 16)] = xl_v[b, pl.ds(16 * h, 16)] * ex[h]
            return cin
        lax.fori_loop(0, OC, _edge, 0)
        pltpu.sync_copy(msg_v, u_sh.at[idx_d], add=True)
        return carry
    lax.fori_loop(0, CH, _chunk, 0)
    plsc.subcore_barrier()
    _writeout(u_sh, u_out, c, row0)


@functools.partial(
    pl.kernel, mesh=_MESH,
    compiler_params=pltpu.CompilerParams(use_tc_tiling_on_sc=False),
    out_type=jax.ShapeDtypeStruct((NC, NPAD, 16), jnp.float32),
    scratch_types=[
        pltpu.VMEM((OC,), jnp.int32),
        pltpu.VMEM((OC,), jnp.int32),
        pltpu.VMEM((OC, 16), jnp.float32),
        pltpu.VMEM((OC, 16), jnp.float32),
        pltpu.VMEM((OC, 16), jnp.float32),
        pltpu.VMEM((16, 16), jnp.float32),
        pltpu.VMEM_SHARED((NPAD, 16), jnp.float32),
        pltpu.SemaphoreType.DMA,
    ],
)
def _edge2_denom(src_hbm, dst_hbm, as2_hbm, ad2_hbm, d_out,
                 idx_s, idx_d, as_v, ad_v, ex_v, zbuf16, d_sh, sem):
    c = lax.axis_index("c")
    s = lax.axis_index("s")
    w = c * NS + s
    row0 = s * RPT
    _zero_zbuf(zbuf16, 16)
    _zero_shared(zbuf16, d_sh, row0)
    plsc.subcore_barrier()

    def _chunk(k, carry):
        base = (w * CH + k) * OC
        pltpu.sync_copy(src_hbm.at[pl.ds(base, OC)], idx_s)
        pltpu.sync_copy(dst_hbm.at[pl.ds(base, OC)], idx_d)
        pltpu.async_copy(as2_hbm.at[idx_s], as_v, sem).wait()
        pltpu.async_copy(ad2_hbm.at[idx_d], ad_v, sem).wait()

        def _edge(b, cin):
            ex_v[b, :] = jnp.exp(_lrelu(as_v[b, :] + ad_v[b, :]))
            return cin
        lax.fori_loop(0, OC, _edge, 0)
        pltpu.sync_copy(ex_v, d_sh.at[idx_d], add=True)
        return carry
    lax.fori_loop(0, CH, _chunk, 0)
    plsc.subcore_barrier()
    _writeout(d_sh, d_out, c, row0)


@functools.partial(
    pl.kernel, mesh=_MESH,
    compiler_params=pltpu.CompilerParams(use_tc_tiling_on_sc=False),
    out_type=jax.ShapeDtypeStruct((NC, NPAD, 128), jnp.float32),
    scratch_types=[
        pltpu.VMEM((4, 32), jnp.int32),
        pltpu.VMEM((OC,), jnp.int32),
        pltpu.VMEM((OC,), jnp.int32),
        pltpu.VMEM((OC, 16), jnp.float32),
        pltpu.VMEM((OC, 16), jnp.float32),
        pltpu.VMEM((OC, 16), jnp.float32),
        pltpu.VMEM((OC, 16), jnp.float32),
        pltpu.VMEM((32, 1024), jnp.float32),
        pltpu.VMEM((32, 128), jnp.float32),
        pltpu.VMEM((16, 128), jnp.float32),
        pltpu.VMEM_SHARED((NPAD, 128), jnp.float32),
        pltpu.SemaphoreType.DMA,
    ],
)
def _edge2_msg(src_hbm, dst_hbm, xl2_hbm, as2_hbm, ad2_hbm, r2_hbm, o_out,
               idx_d2, idx_s, idx_d, as_v, ad_v, r_v, at_v,
               xl_v, msg_v, zbuf, o_sh, sem):
    c = lax.axis_index("c")
    s = lax.axis_index("s")
    w = c * NS + s
    row0 = s * RPT
    _zero_zbuf(zbuf, 128)
    _zero_shared(zbuf, o_sh, row0)
    plsc.subcore_barrier()

    def _chunk(k, carry):
        base = (w * CH + k) * OC
        pltpu.sync_copy(src_hbm.at[pl.ds(base, OC)], idx_s)
        pltpu.sync_copy(dst_hbm.at[pl.ds(base, OC)], idx_d)
        for sub in range(4):
            pltpu.sync_copy(dst_hbm.at[pl.ds(base + 32 * sub, 32)],
                            idx_d2.at[sub])
        pltpu.async_copy(as2_hbm.at[idx_s], as_v, sem).wait()
        pltpu.async_copy(ad2_hbm.at[idx_d], ad_v, sem).wait()
        pltpu.async_copy(r2_hbm.at[idx_d], r_v, sem).wait()

        def _att(b, cin):
            at_v[b, :] = jnp.exp(_lrelu(as_v[b, :] + ad_v[b, :])) * r_v[b, :]
            return cin
        lax.fori_loop(0, OC, _att, 0)

        for sub in range(4):
            pltpu.async_copy(xl2_hbm.at[idx_s.at[pl.ds(32 * sub, 32)]],
                             xl_v, sem).wait()

            def _edge(b, cin):
                bb = 32 * sub + b
                atr = at_v[bb, :]
                att = [atr[h] for h in range(NH)]
                for j in range(8):
                    acc = xl_v[b, pl.ds(16 * j, 16)] * att[0]
                    for h in range(1, NH):
                        acc = acc + xl_v[b, pl.ds(128 * h + 16 * j, 16)] * att[h]
                    msg_v[b, pl.ds(16 * j, 16)] = acc
                return cin
            lax.fori_loop(0, 32, _edge, 0)
            pltpu.sync_copy(msg_v, o_sh.at[idx_d2.at[sub]], add=True)
        return carry
    lax.fori_loop(0, CH, _chunk, 0)
    plsc.subcore_barrier()
    _writeout(o_sh, o_out, c, row0)


def _dup16(m):
    eye = jnp.eye(NH, dtype=jnp.float32)
    blk = m[:, :, None] * eye[:, None, :]
    blk = jnp.concatenate([blk, blk], axis=-1)
    blk = jnp.pad(blk, ((0, 0), (0, 8), (0, 0)))
    return blk.reshape(128, 16)


def _dup2(m):
    eye = jnp.eye(NH, dtype=jnp.float32)
    blk = m[:, :, None] * eye[:, None, :]
    blk = jnp.concatenate([blk, blk], axis=-1)
    return blk.reshape(1024, 16)


def kernel(x, edge_index, W1, att_src1, att_dst1, bias1, W2, att_src2,
           att_dst2, bias2):
    f32 = jnp.float32
    cols = np.arange(64).reshape(8, 8)
    cols = (cols // 8 * 16 + cols % 8).reshape(-1)
    w1p = jnp.zeros((DIN, 128), f32).at[:, cols].set(W1)
    w2p = jnp.zeros((128, 1024), f32).at[cols].set(W2)
    b1p = jnp.zeros((128,), f32).at[cols].set(bias1)
    m1s, m1d = _dup16(att_src1), _dup16(att_dst1)
    m2s, m2d = _dup2(att_src2), _dup2(att_dst2)
    s2 = np.zeros((128, 128), np.float32)
    ones = np.zeros((BLK, 128), np.float32)
    for h in range(NH):
        s2[h * 16 + 8, h * 16:h * 16 + 8] = 1.0
        ones[:, h * 16 + 8] = 1.0
    s2, ones = jnp.asarray(s2), jnp.asarray(ones)

    xp = jnp.pad(x, ((0, NPAD - NNODES), (0, 0)))
    loop = jnp.arange(NNODES, dtype=edge_index.dtype)
    pad = jnp.full((EPAD - ETOT,), NNODES, dtype=edge_index.dtype)
    src = jnp.concatenate([edge_index[0], loop, pad])
    dst = jnp.concatenate([edge_index[1], loop, pad])

    nblk = NPAD // BLK
    xl1, as1, ad1 = pl.pallas_call(
        _k1_body,
        grid=(nblk,),
        in_specs=[
            pl.BlockSpec((BLK, DIN), lambda i: (i, 0)),
            pl.BlockSpec((DIN, 128), lambda i: (0, 0)),
            pl.BlockSpec((128, 16), lambda i: (0, 0)),
            pl.BlockSpec((128, 16), lambda i: (0, 0)),
            pl.BlockSpec((BLK, 128), lambda i: (0, 0)),
        ],
        out_specs=[
            pl.BlockSpec((BLK, 128), lambda i: (i, 0)),
            pl.BlockSpec((BLK, 16), lambda i: (i, 0)),
            pl.BlockSpec((BLK, 16), lambda i: (i, 0)),
        ],
        out_shape=[
            jax.ShapeDtypeStruct((NPAD, 128), f32),
            jax.ShapeDtypeStruct((NPAD, 16), f32),
            jax.ShapeDtypeStruct((NPAD, 16), f32),
        ],
    )(xp, w1p, m1s, m1d, ones)

    u_p = _edge1(src, dst, xl1, as1, ad1)

    xl2, as2, ad2 = pl.pallas_call(
        _k3_body,
        grid=(nblk,),
        in_specs=[
            pl.BlockSpec((NC, BLK, 128), lambda i: (0, i, 0)),
            pl.BlockSpec((128, 128), lambda i: (0, 0)),
            pl.BlockSpec((128,), lambda i: (0,)),
            pl.BlockSpec((128, 1024), lambda i: (0, 0)),
            pl.BlockSpec((1024, 16), lambda i: (0, 0)),
            pl.BlockSpec((1024, 16), lambda i: (0, 0)),
        ],
        out_specs=[
            pl.BlockSpec((BLK, 1024), lambda i: (i, 0)),
            pl.BlockSpec((BLK, 16), lambda i: (i, 0)),
            pl.BlockSpec((BLK, 16), lambda i: (i, 0)),
        ],
        out_shape=[
            jax.ShapeDtypeStruct((NPAD, 1024), f32),
            jax.ShapeDtypeStruct((NPAD, 16), f32),
            jax.ShapeDtypeStruct((NPAD, 16), f32),
        ],
    )(u_p, s2, b1p, w2p, m2s, m2d)

    d2_p = _edge2_denom(src, dst, as2, ad2)

    r2 = pl.pallas_call(
        _k4b_body,
        grid=(4,),
        in_specs=[pl.BlockSpec((NC, NPAD // 4, 16), lambda i: (0, i, 0))],
        out_specs=pl.BlockSpec((NPAD // 4, 16), lambda i: (i, 0)),
        out_shape=jax.ShapeDtypeStruct((NPAD, 16), f32),
    )(d2_p)

    o_p = _edge2_msg(src, dst, xl2, as2, ad2, r2)

    out = pl.pallas_call(
        _k6_body,
        grid=(4,),
        in_specs=[
            pl.BlockSpec((NC, NPAD // 4, 128), lambda i: (0, i, 0)),
            pl.BlockSpec((128,), lambda i: (0,)),
        ],
        out_specs=pl.BlockSpec((NPAD // 4, 128), lambda i: (i, 0)),
        out_shape=jax.ShapeDtypeStruct((NPAD, 128), f32),
    )(o_p, bias2)

    return out[:NNODES]


# speedup vs baseline: 35.9681x; 1.5279x over previous
import functools

import jax
import jax.numpy as jnp
import numpy as np
from jax import lax
from jax.experimental import pallas as pl
from jax.experimental.pallas import tpu as pltpu
from jax.experimental.pallas import tpu_sc as plsc

NNODES = 10000
NPAD = 10240
DIN = 128
NH = 8
ETOT = 330000
NC, NS = 2, 16
NW = NC * NS
OC = 128
CH = 82
EPAD = NW * CH * OC
ESTORE = EPAD + OC
RPT = NPAD // NS
BLK = 512


def _lrelu(a):
    return jnp.where(a > 0, a, 0.2 * a)


def _k1_body(x_ref, w1p_ref, m1s_ref, m1d_ref, ones_ref, xl_ref, as_ref,
             ad_ref):
    x = x_ref[...]
    xl = jnp.dot(x, w1p_ref[...], preferred_element_type=jnp.float32)
    xl_ref[...] = xl + ones_ref[...]
    as_ref[...] = jnp.dot(xl, m1s_ref[...], preferred_element_type=jnp.float32)
    ad_ref[...] = jnp.dot(xl, m1d_ref[...], preferred_element_type=jnp.float32)


def _k3_body(up_ref, s2_ref, b1_ref, w2p_ref, m2s_ref, m2d_ref,
             xl2_ref, as2_ref, ad2_ref):
    u = up_ref[0] + up_ref[1]
    d128 = jnp.dot(u, s2_ref[...], preferred_element_type=jnp.float32)
    hpre = u / (d128 + 1e-16) + b1_ref[...]
    h = jnp.where(hpre > 0, hpre, 0.2 * (jnp.exp(hpre) - 1.0))
    xl2 = jnp.dot(h, w2p_ref[...], preferred_element_type=jnp.float32)
    xl2_ref[...] = xl2
    as2_ref[...] = jnp.dot(xl2, m2s_ref[...], preferred_element_type=jnp.float32)
    ad2_ref[...] = jnp.dot(xl2, m2d_ref[...], preferred_element_type=jnp.float32)


def _k4b_body(dp_ref, r_ref):
    r_ref[...] = 1.0 / (dp_ref[0] + dp_ref[1] + 1e-16)


def _k6_body(op_ref, b2_ref, o_ref):
    o_ref[...] = (op_ref[0] + op_ref[1]) * 0.125 + b2_ref[...]


_MESH = plsc.VectorSubcoreMesh(core_axis_name="c", subcore_axis_name="s")


def _zero_zbuf(zbuf, cols):
    z16 = jnp.zeros((16,), jnp.float32)
    for i in range(16):
        for j in range(cols // 16):
            zbuf[i, pl.ds(16 * j, 16)] = z16


def _zero_shared(zbuf, sh_ref, row0):
    def _z(k, carry):
        pltpu.sync_copy(zbuf, sh_ref.at[pl.ds(row0 + 16 * k, 16)])
        return carry
    lax.fori_loop(0, RPT // 16, _z, 0)


def _writeout(sh_ref, out_ref, core, row0):
    def _w(k, carry):
        r = row0 + 16 * k
        pltpu.sync_copy(sh_ref.at[pl.ds(r, 16)], out_ref.at[core, pl.ds(r, 16)])
        return carry
    lax.fori_loop(0, RPT // 16, _w, 0)


@functools.partial(
    pl.kernel, mesh=_MESH,
    compiler_params=pltpu.CompilerParams(use_tc_tiling_on_sc=False),
    out_type=jax.ShapeDtypeStruct((NC, NPAD, 128), jnp.float32),
    scratch_types=[
        [pltpu.VMEM((OC,), jnp.int32)] * 2,
        [pltpu.VMEM((OC,), jnp.int32)] * 2,
        [pltpu.VMEM((OC, 16), jnp.float32)] * 2,
        [pltpu.VMEM((OC, 16), jnp.float32)] * 2,
        [pltpu.VMEM((OC, 128), jnp.float32)] * 2,
        pltpu.VMEM_SHARED((NPAD, 128), jnp.float32),
        [pltpu.SemaphoreType.DMA] * 2,
        pltpu.SemaphoreType.DMA,
    ],
)
def _edge1(src_hbm, dst_hbm, xl1_hbm, as1_hbm, ad1_hbm, u_out,
           idx_s, idx_d, as_v, ad_v, xl_v, u_sh, semg, semi):
    c = lax.axis_index("c")
    s = lax.axis_index("s")
    w = c * NS + s
    row0 = s * RPT
    _zero_zbuf(xl_v[0], 128)
    _zero_shared(xl_v[0].at[pl.ds(0, 16)], u_sh, row0)
    plsc.subcore_barrier()

    def _issue_idx(k, p):
        base = (w * CH + k) * OC
        pltpu.async_copy(src_hbm.at[pl.ds(base, OC)], idx_s[p], semi)
        pltpu.async_copy(dst_hbm.at[pl.ds(base, OC)], idx_d[p], semi)

    def _drain_idx(p):
        pltpu.make_async_copy(src_hbm.at[pl.ds(0, OC)], idx_s[p], semi).wait()
        pltpu.make_async_copy(dst_hbm.at[pl.ds(0, OC)], idx_d[p], semi).wait()

    def _issue_g(p):
        pltpu.async_copy(as1_hbm.at[idx_s[p]], as_v[p], semg[p])
        pltpu.async_copy(ad1_hbm.at[idx_d[p]], ad_v[p], semg[p])
        pltpu.async_copy(xl1_hbm.at[idx_s[p]], xl_v[p], semg[p])

    def _drain_g(p):
        pltpu.make_async_copy(as1_hbm.at[idx_s[p]], as_v[p], semg[p]).wait()
        pltpu.make_async_copy(ad1_hbm.at[idx_d[p]], ad_v[p], semg[p]).wait()
        pltpu.make_async_copy(xl1_hbm.at[idx_s[p]], xl_v[p], semg[p]).wait()

    def _process(k, p):
        _issue_idx(k + 1, 1 - p)
        _drain_g(p)

        def _edge(b, cin):
            ex = jnp.exp(_lrelu(as_v[p][b, :] + ad_v[p][b, :]))
            for h in range(NH):
                xl_v[p][b, pl.ds(16 * h, 16)] = (
                    xl_v[p][b, pl.ds(16 * h, 16)] * ex[h])
            return cin
        lax.fori_loop(0, OC // 2, _edge, 0)
        _drain_idx(1 - p)
        _issue_g(1 - p)
        lax.fori_loop(OC // 2, OC, _edge, 0)
        pltpu.sync_copy(xl_v[p], u_sh.at[idx_d[p]], add=True)

    _issue_idx(0, 0)
    _drain_idx(0)
    _issue_g(0)

    def _pair(i, carry):
        _process(2 * i, 0)
        _process(2 * i + 1, 1)
        return carry
    lax.fori_loop(0, CH // 2, _pair, 0)
    _drain_g(0)
    plsc.subcore_barrier()
    _writeout(u_sh, u_out, c, row0)


@functools.partial(
    pl.kernel, mesh=_MESH,
    compiler_params=pltpu.CompilerParams(use_tc_tiling_on_sc=False),
    out_type=jax.ShapeDtypeStruct((NC, NPAD, 16), jnp.float32),
    scratch_types=[
        [pltpu.VMEM((OC,), jnp.int32)] * 2,
        [pltpu.VMEM((OC,), jnp.int32)] * 2,
        [pltpu.VMEM((OC, 16), jnp.float32)] * 2,
        [pltpu.VMEM((OC, 16), jnp.float32)] * 2,
        pltpu.VMEM((OC, 16), jnp.float32),
        pltpu.VMEM((16, 16), jnp.float32),
        pltpu.VMEM_SHARED((NPAD, 16), jnp.float32),
        [pltpu.SemaphoreType.DMA] * 2,
        pltpu.SemaphoreType.DMA,
    ],
)
def _edge2_denom(src_hbm, dst_hbm, as2_hbm, ad2_hbm, d_out,
                 idx_s, idx_d, as_v, ad_v, ex_v, zbuf16, d_sh, semg, semi):
    c = lax.axis_index("c")
    s = lax.axis_index("s")
    w = c * NS + s
    row0 = s * RPT
    _zero_zbuf(zbuf16, 16)
    _zero_shared(zbuf16, d_sh, row0)
    plsc.subcore_barrier()

    def _issue_idx(k, p):
        base = (w * CH + k) * OC
        pltpu.async_copy(src_hbm.at[pl.ds(base, OC)], idx_s[p], semi)
        pltpu.async_copy(dst_hbm.at[pl.ds(base, OC)], idx_d[p], semi)

    def _drain_idx(p):
        pltpu.make_async_copy(src_hbm.at[pl.ds(0, OC)], idx_s[p], semi).wait()
        pltpu.make_async_copy(dst_hbm.at[pl.ds(0, OC)], idx_d[p], semi).wait()

    def _issue_g(p):
        pltpu.async_copy(as2_hbm.at[idx_s[p]], as_v[p], semg[p])
        pltpu.async_copy(ad2_hbm.at[idx_d[p]], ad_v[p], semg[p])

    def _drain_g(p):
        pltpu.make_async_copy(as2_hbm.at[idx_s[p]], as_v[p], semg[p]).wait()
        pltpu.make_async_copy(ad2_hbm.at[idx_d[p]], ad_v[p], semg[p]).wait()

    def _process(k, p):
        _issue_idx(k + 1, 1 - p)
        _drain_g(p)

        def _edge(b, cin):
            ex_v[b, :] = jnp.exp(_lrelu(as_v[p][b, :] + ad_v[p][b, :]))
            return cin
        lax.fori_loop(0, OC // 2, _edge, 0)
        _drain_idx(1 - p)
        _issue_g(1 - p)
        lax.fori_loop(OC // 2, OC, _edge, 0)
        pltpu.sync_copy(ex_v, d_sh.at[idx_d[p]], add=True)

    _issue_idx(0, 0)
    _drain_idx(0)
    _issue_g(0)

    def _pair(i, carry):
        _process(2 * i, 0)
        _process(2 * i + 1, 1)
        return carry
    lax.fori_loop(0, CH // 2, _pair, 0)
    _drain_g(0)
    plsc.subcore_barrier()
    _writeout(d_sh, d_out, c, row0)


@functools.partial(
    pl.kernel, mesh=_MESH,
    compiler_params=pltpu.CompilerParams(use_tc_tiling_on_sc=False),
    out_type=jax.ShapeDtypeStruct((NC, NPAD, 128), jnp.float32),
    scratch_types=[
        [pltpu.VMEM((8, 16), jnp.int32)] * 2,
        [pltpu.VMEM((OC,), jnp.int32)] * 2,
        [pltpu.VMEM((OC,), jnp.int32)] * 2,
        [pltpu.VMEM((OC, 16), jnp.float32)] * 2,
        [pltpu.VMEM((OC, 16), jnp.float32)] * 2,
        [pltpu.VMEM((OC, 16), jnp.float32)] * 2,
        [pltpu.VMEM((16, 1024), jnp.float32)] * 2,
        pltpu.VMEM((16, 128), jnp.float32),
        pltpu.VMEM_SHARED((NPAD, 128), jnp.float32),
        [pltpu.SemaphoreType.DMA] * 2,
        [pltpu.SemaphoreType.DMA] * 2,
        pltpu.SemaphoreType.DMA,
    ],
)
def _edge2_msg(src_hbm, dst_hbm, xl2_hbm, as2_hbm, ad2_hbm, r2_hbm, o_out,
               idx_d2, idx_s, idx_d, as_v, ad_v, r_v,
               xl_v, msg_v, o_sh, semg, semx, semi):
    c = lax.axis_index("c")
    s = lax.axis_index("s")
    w = c * NS + s
    row0 = s * RPT
    _zero_zbuf(msg_v, 128)
    _zero_shared(msg_v, o_sh, row0)
    plsc.subcore_barrier()

    def _issue_idx(k, p):
        base = (w * CH + k) * OC
        pltpu.async_copy(src_hbm.at[pl.ds(base, OC)], idx_s[p], semi)
        pltpu.async_copy(dst_hbm.at[pl.ds(base, OC)], idx_d[p], semi)
        for sub in range(8):
            pltpu.async_copy(dst_hbm.at[pl.ds(base + 16 * sub, 16)],
                             idx_d2[p].at[sub], semi)

    def _drain_idx(p):
        pltpu.make_async_copy(src_hbm.at[pl.ds(0, OC)], idx_s[p], semi).wait()
        pltpu.make_async_copy(dst_hbm.at[pl.ds(0, OC)], idx_d[p], semi).wait()
        for sub in range(8):
            pltpu.make_async_copy(dst_hbm.at[pl.ds(0, 16)],
                                  idx_d2[p].at[sub], semi).wait()

    def _issue_g(p):
        pltpu.async_copy(as2_hbm.at[idx_s[p]], as_v[p], semg[p])
        pltpu.async_copy(ad2_hbm.at[idx_d[p]], ad_v[p], semg[p])
        pltpu.async_copy(r2_hbm.at[idx_d[p]], r_v[p], semg[p])

    def _drain_g(p):
        pltpu.make_async_copy(as2_hbm.at[idx_s[p]], as_v[p], semg[p]).wait()
        pltpu.make_async_copy(ad2_hbm.at[idx_d[p]], ad_v[p], semg[p]).wait()
        pltpu.make_async_copy(r2_hbm.at[idx_d[p]], r_v[p], semg[p]).wait()

    def _issue_xl(p, sub, xp):
        pltpu.async_copy(xl2_hbm.at[idx_s[p].at[pl.ds(16 * sub, 16)]],
                         xl_v[xp], semx[xp])

    def _drain_xl(xp):
        pltpu.make_async_copy(xl2_hbm.at[idx_s[0].at[pl.ds(0, 16)]],
                              xl_v[xp], semx[xp]).wait()

    def _process(k, p):
        _issue_idx(k + 1, 1 - p)
        _drain_g(p)

        def _att(b, cin):
            as_v[p][b, :] = (jnp.exp(_lrelu(as_v[p][b, :] + ad_v[p][b, :]))
                             * r_v[p][b, :])
            return cin
        lax.fori_loop(0, OC, _att, 0)

        for sub in range(8):
            xp = sub % 2
            _drain_xl(xp)
            if sub < 7:
                _issue_xl(p, sub + 1, 1 - xp)
            if sub == 1:
                _drain_idx(1 - p)
                _issue_g(1 - p)

            def _edge(b, cin):
                bb = 16 * sub + b
                atr = as_v[p][bb, :]
                att = [atr[h] for h in range(NH)]
                for j in range(8):
                    acc = xl_v[xp][b, pl.ds(16 * j, 16)] * att[0]
                    for h in range(1, NH):
                        acc = acc + (xl_v[xp][b, pl.ds(128 * h + 16 * j, 16)]
                                     * att[h])
                    msg_v[b, pl.ds(16 * j, 16)] = acc
                return cin
            lax.fori_loop(0, 16, _edge, 0)
            if sub == 7:
                _issue_xl(1 - p, 0, 1 - xp)
            pltpu.sync_copy(msg_v, o_sh.at[idx_d2[p].at[sub]], add=True)

    _issue_idx(0, 0)
    _drain_idx(0)
    _issue_g(0)
    _issue_xl(0, 0, 0)

    def _pair(i, carry):
        _process(2 * i, 0)
        _process(2 * i + 1, 1)
        return carry
    lax.fori_loop(0, CH // 2, _pair, 0)
    _drain_g(0)
    _drain_xl(0)
    plsc.subcore_barrier()
    _writeout(o_sh, o_out, c, row0)


def _dup16(m):
    eye = jnp.eye(NH, dtype=jnp.float32)
    blk = m[:, :, None] * eye[:, None, :]
    blk = jnp.concatenate([blk, blk], axis=-1)
    blk = jnp.pad(blk, ((0, 0), (0, 8), (0, 0)))
    return blk.reshape(128, 16)


def _dup2(m):
    eye = jnp.eye(NH, dtype=jnp.float32)
    blk = m[:, :, None] * eye[:, None, :]
    blk = jnp.concatenate([blk, blk], axis=-1)
    return blk.reshape(1024, 16)


def kernel(x, edge_index, W1, att_src1, att_dst1, bias1, W2, att_src2,
           att_dst2, bias2):
    f32 = jnp.float32
    cols = np.arange(64).reshape(8, 8)
    cols = (cols // 8 * 16 + cols % 8).reshape(-1)
    w1p = jnp.zeros((DIN, 128), f32).at[:, cols].set(W1)
    w2p = jnp.zeros((128, 1024), f32).at[cols].set(W2)
    b1p = jnp.zeros((128,), f32).at[cols].set(bias1)
    m1s, m1d = _dup16(att_src1), _dup16(att_dst1)
    m2s, m2d = _dup2(att_src2), _dup2(att_dst2)
    s2 = np.zeros((128, 128), np.float32)
    ones = np.zeros((BLK, 128), np.float32)
    for h in range(NH):
        s2[h * 16 + 8, h * 16:h * 16 + 8] = 1.0
        ones[:, h * 16 + 8] = 1.0
    s2, ones = jnp.asarray(s2), jnp.asarray(ones)

    xp = jnp.pad(x, ((0, NPAD - NNODES), (0, 0)))
    loop = jnp.arange(NNODES, dtype=edge_index.dtype)
    pad = jnp.full((ESTORE - ETOT,), NNODES, dtype=edge_index.dtype)
    src = jnp.concatenate([edge_index[0], loop, pad])
    dst = jnp.concatenate([edge_index[1], loop, pad])

    nblk = NPAD // BLK
    xl1, as1, ad1 = pl.pallas_call(
        _k1_body,
        grid=(nblk,),
        in_specs=[
            pl.BlockSpec((BLK, DIN), lambda i: (i, 0)),
            pl.BlockSpec((DIN, 128), lambda i: (0, 0)),
            pl.BlockSpec((128, 16), lambda i: (0, 0)),
            pl.BlockSpec((128, 16), lambda i: (0, 0)),
            pl.BlockSpec((BLK, 128), lambda i: (0, 0)),
        ],
        out_specs=[
            pl.BlockSpec((BLK, 128), lambda i: (i, 0)),
            pl.BlockSpec((BLK, 16), lambda i: (i, 0)),
            pl.BlockSpec((BLK, 16), lambda i: (i, 0)),
        ],
        out_shape=[
            jax.ShapeDtypeStruct((NPAD, 128), f32),
            jax.ShapeDtypeStruct((NPAD, 16), f32),
            jax.ShapeDtypeStruct((NPAD, 16), f32),
        ],
    )(xp, w1p, m1s, m1d, ones)

    u_p = _edge1(src, dst, xl1, as1, ad1)

    xl2, as2, ad2 = pl.pallas_call(
        _k3_body,
        grid=(nblk,),
        in_specs=[
            pl.BlockSpec((NC, BLK, 128), lambda i: (0, i, 0)),
            pl.BlockSpec((128, 128), lambda i: (0, 0)),
            pl.BlockSpec((128,), lambda i: (0,)),
            pl.BlockSpec((128, 1024), lambda i: (0, 0)),
            pl.BlockSpec((1024, 16), lambda i: (0, 0)),
            pl.BlockSpec((1024, 16), lambda i: (0, 0)),
        ],
        out_specs=[
            pl.BlockSpec((BLK, 1024), lambda i: (i, 0)),
            pl.BlockSpec((BLK, 16), lambda i: (i, 0)),
            pl.BlockSpec((BLK, 16), lambda i: (i, 0)),
        ],
        out_shape=[
            jax.ShapeDtypeStruct((NPAD, 1024), f32),
            jax.ShapeDtypeStruct((NPAD, 16), f32),
            jax.ShapeDtypeStruct((NPAD, 16), f32),
        ],
    )(u_p, s2, b1p, w2p, m2s, m2d)

    d2_p = _edge2_denom(src, dst, as2, ad2)

    r2 = pl.pallas_call(
        _k4b_body,
        grid=(4,),
        in_specs=[pl.BlockSpec((NC, NPAD // 4, 16), lambda i: (0, i, 0))],
        out_specs=pl.BlockSpec((NPAD // 4, 16), lambda i: (i, 0)),
        out_shape=jax.ShapeDtypeStruct((NPAD, 16), f32),
    )(d2_p)

    o_p = _edge2_msg(src, dst, xl2, as2, ad2, r2)

    out = pl.pallas_call(
        _k6_body,
        grid=(4,),
        in_specs=[
            pl.BlockSpec((NC, NPAD // 4, 128), lambda i: (0, i, 0)),
            pl.BlockSpec((128,), lambda i: (0,)),
        ],
        out_specs=pl.BlockSpec((NPAD // 4, 128), lambda i: (i, 0)),
        out_shape=jax.ShapeDtypeStruct((NPAD, 128), f32),
    )(o_p, bias2)

    return out[:NNODES]


# speedup vs baseline: 38.8410x; 1.0799x over previous
import functools

import jax
import jax.numpy as jnp
import numpy as np
from jax import lax
from jax.experimental import pallas as pl
from jax.experimental.pallas import tpu as pltpu
from jax.experimental.pallas import tpu_sc as plsc

NNODES = 10000
NPAD = 10240
DIN = 128
NH = 8
ETOT = 330000
NC, NS = 2, 16
NW = NC * NS
OC = 128
CH = 82
EPAD = NW * CH * OC
ESTORE = EPAD + OC
RPT = NPAD // NS
BLK = 512


def _lrelu(a):
    return jnp.where(a > 0, a, 0.2 * a)


def _vcopy(src_ref, dst_ref, n):
    for j in range(n // 16):
        dst_ref[pl.ds(16 * j, 16)] = src_ref[pl.ds(16 * j, 16)]


def _k1_body(x_ref, w1p_ref, m1s_ref, m1d_ref, ones_ref, xl_ref, as_ref,
             ad_ref):
    x = x_ref[...]
    xl = jnp.dot(x, w1p_ref[...], preferred_element_type=jnp.float32)
    xl_ref[...] = xl + ones_ref[...]
    as_ref[...] = jnp.dot(xl, m1s_ref[...], preferred_element_type=jnp.float32)
    ad_ref[...] = jnp.dot(xl, m1d_ref[...], preferred_element_type=jnp.float32)


def _k3_body(up_ref, s2_ref, b1_ref, w2p_ref, m2s_ref, m2d_ref,
             xl2_ref, as2_ref, ad2_ref):
    u = up_ref[0] + up_ref[1]
    d128 = jnp.dot(u, s2_ref[...], preferred_element_type=jnp.float32)
    hpre = u / (d128 + 1e-16) + b1_ref[...]
    h = jnp.where(hpre > 0, hpre, 0.2 * (jnp.exp(hpre) - 1.0))
    xl2 = jnp.dot(h, w2p_ref[...], preferred_element_type=jnp.float32)
    xl2_ref[...] = xl2
    as2_ref[...] = jnp.dot(xl2, m2s_ref[...], preferred_element_type=jnp.float32)
    ad2_ref[...] = jnp.dot(xl2, m2d_ref[...], preferred_element_type=jnp.float32)


def _k4b_body(dp_ref, r_ref):
    r_ref[...] = 1.0 / (dp_ref[0] + dp_ref[1] + 1e-16)


def _k6_body(op_ref, b2_ref, o_ref):
    o_ref[...] = (op_ref[0] + op_ref[1]) * 0.125 + b2_ref[...]


_MESH = plsc.VectorSubcoreMesh(core_axis_name="c", subcore_axis_name="s")


def _zero_zbuf(zbuf, cols):
    z16 = jnp.zeros((16,), jnp.float32)
    for i in range(16):
        for j in range(cols // 16):
            zbuf[i, pl.ds(16 * j, 16)] = z16


def _zero_shared(zbuf, sh_ref, row0):
    def _z(k, carry):
        pltpu.sync_copy(zbuf, sh_ref.at[pl.ds(row0 + 16 * k, 16)])
        return carry
    lax.fori_loop(0, RPT // 16, _z, 0)


def _writeout(sh_ref, out_ref, core, row0):
    def _w(k, carry):
        r = row0 + 16 * k
        pltpu.sync_copy(sh_ref.at[pl.ds(r, 16)], out_ref.at[core, pl.ds(r, 16)])
        return carry
    lax.fori_loop(0, RPT // 16, _w, 0)


@functools.partial(
    pl.kernel, mesh=_MESH,
    compiler_params=pltpu.CompilerParams(use_tc_tiling_on_sc=False),
    out_type=jax.ShapeDtypeStruct((NC, NPAD, 128), jnp.float32),
    scratch_types=[
        [pltpu.VMEM((OC,), jnp.int32)] * 2,
        [pltpu.VMEM((OC,), jnp.int32)] * 2,
        [pltpu.VMEM((OC, 16), jnp.float32)] * 2,
        [pltpu.VMEM((OC, 16), jnp.float32)] * 2,
        [pltpu.VMEM((OC, 128), jnp.float32)] * 2,
        [pltpu.VMEM((OC,), jnp.int32)] * 2,
        pltpu.VMEM_SHARED((NPAD, 128), jnp.float32),
        [pltpu.SemaphoreType.DMA] * 2,
        pltpu.SemaphoreType.DMA,
        [pltpu.SemaphoreType.DMA] * 2,
    ],
)
def _edge1(src_hbm, dst_hbm, xl1_hbm, as1_hbm, ad1_hbm, u_out,
           idx_s, idx_d, as_v, ad_v, xl_v, idx_dsc, u_sh, semg, semi, semsc):
    c = lax.axis_index("c")
    s = lax.axis_index("s")
    w = c * NS + s
    row0 = s * RPT
    _zero_zbuf(xl_v[0], 128)
    _zero_shared(xl_v[0].at[pl.ds(0, 16)], u_sh, row0)
    plsc.subcore_barrier()

    def _issue_sc(p):
        _vcopy(idx_d[p], idx_dsc[p], OC)
        pltpu.async_copy(xl_v[p], u_sh.at[idx_dsc[p]], semsc[p], add=True)

    def _drain_sc(p):
        pltpu.make_async_copy(xl1_hbm.at[pl.ds(0, OC)], xl_v[p],
                              semsc[p]).wait()

    def _issue_idx(k, p):
        base = (w * CH + k) * OC
        pltpu.async_copy(src_hbm.at[pl.ds(base, OC)], idx_s[p], semi)
        pltpu.async_copy(dst_hbm.at[pl.ds(base, OC)], idx_d[p], semi)

    def _drain_idx(p):
        pltpu.make_async_copy(src_hbm.at[pl.ds(0, OC)], idx_s[p], semi).wait()
        pltpu.make_async_copy(dst_hbm.at[pl.ds(0, OC)], idx_d[p], semi).wait()

    def _issue_g(p):
        pltpu.async_copy(as1_hbm.at[idx_s[p]], as_v[p], semg[p])
        pltpu.async_copy(ad1_hbm.at[idx_d[p]], ad_v[p], semg[p])
        pltpu.async_copy(xl1_hbm.at[idx_s[p]], xl_v[p], semg[p])

    def _drain_g(p):
        pltpu.make_async_copy(as1_hbm.at[idx_s[p]], as_v[p], semg[p]).wait()
        pltpu.make_async_copy(ad1_hbm.at[idx_d[p]], ad_v[p], semg[p]).wait()
        pltpu.make_async_copy(xl1_hbm.at[idx_s[p]], xl_v[p], semg[p]).wait()

    def _process(k, p):
        _issue_idx(k + 1, 1 - p)
        _drain_g(p)

        def _edge(b, cin):
            ex = jnp.exp(_lrelu(as_v[p][b, :] + ad_v[p][b, :]))
            for h in range(NH):
                xl_v[p][b, pl.ds(16 * h, 16)] = (
                    xl_v[p][b, pl.ds(16 * h, 16)] * ex[h])
            return cin
        lax.fori_loop(0, OC // 2, _edge, 0)
        _drain_idx(1 - p)
        _drain_sc(1 - p)
        _issue_g(1 - p)
        lax.fori_loop(OC // 2, OC, _edge, 0)
        _issue_sc(p)

    z16i = jnp.zeros((16,), jnp.int32)
    for j in range(NH):
        idx_d[1][pl.ds(16 * j, 16)] = z16i

    def _zrow(b, cin):
        for h in range(NH):
            xl_v[1][b, pl.ds(16 * h, 16)] = jnp.zeros((16,), jnp.float32)
        return cin
    lax.fori_loop(0, OC, _zrow, 0)
    _issue_sc(1)

    _issue_idx(0, 0)
    _drain_idx(0)
    _issue_g(0)

    def _pair(i, carry):
        _process(2 * i, 0)
        _process(2 * i + 1, 1)
        return carry
    lax.fori_loop(0, CH // 2, _pair, 0)
    _drain_g(0)
    _drain_sc(1)
    plsc.subcore_barrier()
    _writeout(u_sh, u_out, c, row0)


@functools.partial(
    pl.kernel, mesh=_MESH,
    compiler_params=pltpu.CompilerParams(use_tc_tiling_on_sc=False),
    out_type=(jax.ShapeDtypeStruct((NC, NPAD, 16), jnp.float32),
              jax.ShapeDtypeStruct((ESTORE, 16), jnp.float32)),
    scratch_types=[
        [pltpu.VMEM((OC,), jnp.int32)] * 2,
        [pltpu.VMEM((OC,), jnp.int32)] * 2,
        [pltpu.VMEM((OC, 16), jnp.float32)] * 2,
        [pltpu.VMEM((OC, 16), jnp.float32)] * 2,
        [pltpu.VMEM((OC, 16), jnp.float32)] * 2,
        [pltpu.VMEM((OC,), jnp.int32)] * 2,
        pltpu.VMEM((16, 16), jnp.float32),
        pltpu.VMEM_SHARED((NPAD, 16), jnp.float32),
        [pltpu.SemaphoreType.DMA] * 2,
        pltpu.SemaphoreType.DMA,
        [pltpu.SemaphoreType.DMA] * 2,
        [pltpu.SemaphoreType.DMA] * 2,
    ],
)
def _edge2_denom(src_hbm, dst_hbm, as2_hbm, ad2_hbm, d_out, ex_out,
                 idx_s, idx_d, as_v, ad_v, ex_v, idx_dsc, zbuf16, d_sh, semg,
                 semi, semsc, semw):
    c = lax.axis_index("c")
    s = lax.axis_index("s")
    w = c * NS + s
    row0 = s * RPT
    _zero_zbuf(zbuf16, 16)
    _zero_shared(zbuf16, d_sh, row0)
    plsc.subcore_barrier()

    def _issue_idx(k, p):
        base = (w * CH + k) * OC
        pltpu.async_copy(src_hbm.at[pl.ds(base, OC)], idx_s[p], semi)
        pltpu.async_copy(dst_hbm.at[pl.ds(base, OC)], idx_d[p], semi)

    def _drain_idx(p):
        pltpu.make_async_copy(src_hbm.at[pl.ds(0, OC)], idx_s[p], semi).wait()
        pltpu.make_async_copy(dst_hbm.at[pl.ds(0, OC)], idx_d[p], semi).wait()

    def _issue_g(p):
        pltpu.async_copy(as2_hbm.at[idx_s[p]], as_v[p], semg[p])
        pltpu.async_copy(ad2_hbm.at[idx_d[p]], ad_v[p], semg[p])

    def _drain_g(p):
        pltpu.make_async_copy(as2_hbm.at[idx_s[p]], as_v[p], semg[p]).wait()
        pltpu.make_async_copy(ad2_hbm.at[idx_d[p]], ad_v[p], semg[p]).wait()

    def _issue_sc(base, p):
        _vcopy(idx_d[p], idx_dsc[p], OC)
        pltpu.async_copy(ex_v[p], d_sh.at[idx_dsc[p]], semsc[p], add=True)
        pltpu.async_copy(ex_v[p], ex_out.at[pl.ds(base, OC)], semw[p])

    def _drain_sc(p):
        pltpu.make_async_copy(as2_hbm.at[pl.ds(0, OC)], ex_v[p],
                              semsc[p]).wait()
        pltpu.make_async_copy(as2_hbm.at[pl.ds(0, OC)], ex_v[p],
                              semw[p]).wait()

    def _process(k, p):
        _issue_idx(k + 1, 1 - p)
        _drain_g(p)
        _drain_sc(p)

        def _edge(b, cin):
            ex_v[p][b, :] = jnp.exp(_lrelu(as_v[p][b, :] + ad_v[p][b, :]))
            return cin
        lax.fori_loop(0, OC // 2, _edge, 0)
        _drain_idx(1 - p)
        _issue_g(1 - p)
        lax.fori_loop(OC // 2, OC, _edge, 0)
        _issue_sc((w * CH + k) * OC, p)

    z16i = jnp.zeros((16,), jnp.int32)
    z16f = jnp.zeros((16,), jnp.float32)
    for pp in range(2):
        for j in range(NH):
            idx_d[pp][pl.ds(16 * j, 16)] = z16i

        def _zrow(b, cin, pp=pp):
            ex_v[pp][b, :] = z16f
            return cin
        lax.fori_loop(0, OC, _zrow, 0)
        _issue_sc(EPAD, pp)

    _issue_idx(0, 0)
    _drain_idx(0)
    _issue_g(0)

    def _pair(i, carry):
        _process(2 * i, 0)
        _process(2 * i + 1, 1)
        return carry
    lax.fori_loop(0, CH // 2, _pair, 0)
    _drain_g(0)
    _drain_sc(0)
    _drain_sc(1)
    plsc.subcore_barrier()
    _writeout(d_sh, d_out, c, row0)


@functools.partial(
    pl.kernel, mesh=_MESH,
    compiler_params=pltpu.CompilerParams(use_tc_tiling_on_sc=False),
    out_type=jax.ShapeDtypeStruct((NC, NPAD, 128), jnp.float32),
    scratch_types=[
        [pltpu.VMEM((OC,), jnp.int32)] * 2,
        [pltpu.VMEM((OC,), jnp.int32)] * 2,
        [pltpu.VMEM((16,), jnp.int32)] * 2,
        [pltpu.VMEM((OC, 16), jnp.float32)] * 2,
        pltpu.VMEM((OC, 16), jnp.float32),
        pltpu.VMEM((OC, 16), jnp.float32),
        [pltpu.VMEM((16, 1024), jnp.float32)] * 2,
        [pltpu.VMEM((16, 128), jnp.float32)] * 2,
        pltpu.VMEM_SHARED((NPAD, 128), jnp.float32),
        [pltpu.SemaphoreType.DMA] * 2,
        [pltpu.SemaphoreType.DMA] * 2,
        pltpu.SemaphoreType.DMA,
        [pltpu.SemaphoreType.DMA] * 2,
    ],
)
def _edge2_msg(src_hbm, dst_hbm, xl2_hbm, ex_hbm, r2_hbm, o_out,
               idx_s, idx_d, idx_dsc, as_v, ad_v, r_v,
               xl_v, msg_v, o_sh, semg, semx, semi, semsc):
    c = lax.axis_index("c")
    s = lax.axis_index("s")
    w = c * NS + s
    row0 = s * RPT
    _zero_zbuf(msg_v[0], 128)
    _zero_zbuf(msg_v[1], 128)
    _zero_shared(msg_v[0], o_sh, row0)
    plsc.subcore_barrier()

    def _issue_idx(k, p):
        base = (w * CH + k) * OC
        pltpu.async_copy(src_hbm.at[pl.ds(base, OC)], idx_s[p], semi)
        pltpu.async_copy(dst_hbm.at[pl.ds(base, OC)], idx_d[p], semi)

    def _drain_idx(p):
        pltpu.make_async_copy(src_hbm.at[pl.ds(0, OC)], idx_s[p], semi).wait()
        pltpu.make_async_copy(dst_hbm.at[pl.ds(0, OC)], idx_d[p], semi).wait()

    def _issue_g(k, p):
        base = (w * CH + k) * OC
        pltpu.async_copy(ex_hbm.at[pl.ds(base, OC)], as_v[p], semg[p])
        pltpu.async_copy(r2_hbm.at[idx_d[p]], r_v, semg[p])

    def _drain_g(p):
        pltpu.make_async_copy(ex_hbm.at[pl.ds(0, OC)], as_v[p], semg[p]).wait()
        pltpu.make_async_copy(r2_hbm.at[idx_d[p]], r_v, semg[p]).wait()

    def _issue_xl(p, sub, xp):
        pltpu.async_copy(xl2_hbm.at[idx_s[p].at[pl.ds(16 * sub, 16)]],
                         xl_v[xp], semx[xp])

    def _drain_xl(xp):
        pltpu.make_async_copy(xl2_hbm.at[idx_s[0].at[pl.ds(0, 16)]],
                              xl_v[xp], semx[xp]).wait()

    def _issue_sc(p, sub, xp):
        idx_dsc[xp][...] = idx_d[p][pl.ds(16 * sub, 16)]
        pltpu.async_copy(msg_v[xp], o_sh.at[idx_dsc[xp]], semsc[xp], add=True)

    def _drain_sc(xp):
        pltpu.make_async_copy(o_out.at[0, pl.ds(0, 16)], msg_v[xp],
                              semsc[xp]).wait()

    def _process(k, p):
        _issue_idx(k + 1, 1 - p)
        _drain_g(p)

        def _att(b, cin):
            as_v[p][b, :] = as_v[p][b, :] * r_v[b, :]
            return cin
        lax.fori_loop(0, OC, _att, 0)

        for sub in range(8):
            xp = sub % 2
            mp = sub % 2
            _drain_xl(xp)
            if sub < 7:
                _issue_xl(p, sub + 1, 1 - xp)
            if sub == 1:
                _drain_idx(1 - p)
                _issue_g(k + 1, 1 - p)
            _drain_sc(mp)

            def _edge(b, cin):
                bb = 16 * sub + b
                atr = as_v[p][bb, :]
                att = [atr[h] for h in range(NH)]
                for j in range(8):
                    acc = xl_v[xp][b, pl.ds(16 * j, 16)] * att[0]
                    for h in range(1, NH):
                        acc = acc + (xl_v[xp][b, pl.ds(128 * h + 16 * j, 16)]
                                     * att[h])
                    msg_v[mp][b, pl.ds(16 * j, 16)] = acc
                return cin
            lax.fori_loop(0, 16, _edge, 0)
            if sub == 7:
                _issue_xl(1 - p, 0, 1 - xp)
            _issue_sc(p, sub, mp)

    z16i = jnp.zeros((16,), jnp.int32)
    for pp in range(2):
        idx_dsc[pp][...] = z16i
        pltpu.async_copy(msg_v[pp], o_sh.at[idx_dsc[pp]], semsc[pp],
                         add=True)

    _issue_idx(0, 0)
    _drain_idx(0)
    _issue_g(0, 0)
    _issue_xl(0, 0, 0)

    def _pair(i, carry):
        _process(2 * i, 0)
        _process(2 * i + 1, 1)
        return carry
    lax.fori_loop(0, CH // 2, _pair, 0)
    _drain_g(0)
    _drain_xl(0)
    _drain_sc(0)
    _drain_sc(1)
    plsc.subcore_barrier()
    _writeout(o_sh, o_out, c, row0)


def _dup16(m):
    eye = jnp.eye(NH, dtype=jnp.float32)
    blk = m[:, :, None] * eye[:, None, :]
    blk = jnp.concatenate([blk, blk], axis=-1)
    blk = jnp.pad(blk, ((0, 0), (0, 8), (0, 0)))
    return blk.reshape(128, 16)


def _dup2(m):
    eye = jnp.eye(NH, dtype=jnp.float32)
    blk = m[:, :, None] * eye[:, None, :]
    blk = jnp.concatenate([blk, blk], axis=-1)
    return blk.reshape(1024, 16)


def kernel(x, edge_index, W1, att_src1, att_dst1, bias1, W2, att_src2,
           att_dst2, bias2):
    f32 = jnp.float32
    cols = np.arange(64).reshape(8, 8)
    cols = (cols // 8 * 16 + cols % 8).reshape(-1)
    w1p = jnp.zeros((DIN, 128), f32).at[:, cols].set(W1)
    w2p = jnp.zeros((128, 1024), f32).at[cols].set(W2)
    b1p = jnp.zeros((128,), f32).at[cols].set(bias1)
    m1s, m1d = _dup16(att_src1), _dup16(att_dst1)
    m2s, m2d = _dup2(att_src2), _dup2(att_dst2)
    s2 = np.zeros((128, 128), np.float32)
    ones = np.zeros((BLK, 128), np.float32)
    for h in range(NH):
        s2[h * 16 + 8, h * 16:h * 16 + 8] = 1.0
        ones[:, h * 16 + 8] = 1.0
    s2, ones = jnp.asarray(s2), jnp.asarray(ones)

    xp = jnp.pad(x, ((0, NPAD - NNODES), (0, 0)))
    loop = jnp.arange(NNODES, dtype=edge_index.dtype)
    pad = jnp.full((ESTORE - ETOT,), NNODES, dtype=edge_index.dtype)
    src = jnp.concatenate([edge_index[0], loop, pad])
    dst = jnp.concatenate([edge_index[1], loop, pad])

    nblk = NPAD // BLK
    xl1, as1, ad1 = pl.pallas_call(
        _k1_body,
        grid=(nblk,),
        in_specs=[
            pl.BlockSpec((BLK, DIN), lambda i: (i, 0)),
            pl.BlockSpec((DIN, 128), lambda i: (0, 0)),
            pl.BlockSpec((128, 16), lambda i: (0, 0)),
            pl.BlockSpec((128, 16), lambda i: (0, 0)),
            pl.BlockSpec((BLK, 128), lambda i: (0, 0)),
        ],
        out_specs=[
            pl.BlockSpec((BLK, 128), lambda i: (i, 0)),
            pl.BlockSpec((BLK, 16), lambda i: (i, 0)),
            pl.BlockSpec((BLK, 16), lambda i: (i, 0)),
        ],
        out_shape=[
            jax.ShapeDtypeStruct((NPAD, 128), f32),
            jax.ShapeDtypeStruct((NPAD, 16), f32),
            jax.ShapeDtypeStruct((NPAD, 16), f32),
        ],
    )(xp, w1p, m1s, m1d, ones)

    u_p = _edge1(src, dst, xl1, as1, ad1)

    xl2, as2, ad2 = pl.pallas_call(
        _k3_body,
        grid=(nblk,),
        in_specs=[
            pl.BlockSpec((NC, BLK, 128), lambda i: (0, i, 0)),
            pl.BlockSpec((128, 128), lambda i: (0, 0)),
            pl.BlockSpec((128,), lambda i: (0,)),
            pl.BlockSpec((128, 1024), lambda i: (0, 0)),
            pl.BlockSpec((1024, 16), lambda i: (0, 0)),
            pl.BlockSpec((1024, 16), lambda i: (0, 0)),
        ],
        out_specs=[
            pl.BlockSpec((BLK, 1024), lambda i: (i, 0)),
            pl.BlockSpec((BLK, 16), lambda i: (i, 0)),
            pl.BlockSpec((BLK, 16), lambda i: (i, 0)),
        ],
        out_shape=[
            jax.ShapeDtypeStruct((NPAD, 1024), f32),
            jax.ShapeDtypeStruct((NPAD, 16), f32),
            jax.ShapeDtypeStruct((NPAD, 16), f32),
        ],
    )(u_p, s2, b1p, w2p, m2s, m2d)

    d2_p, ex2 = _edge2_denom(src, dst, as2, ad2)

    r2 = pl.pallas_call(
        _k4b_body,
        grid=(4,),
        in_specs=[pl.BlockSpec((NC, NPAD // 4, 16), lambda i: (0, i, 0))],
        out_specs=pl.BlockSpec((NPAD // 4, 16), lambda i: (i, 0)),
        out_shape=jax.ShapeDtypeStruct((NPAD, 16), f32),
    )(d2_p)

    o_p = _edge2_msg(src, dst, xl2, ex2, r2)

    out = pl.pallas_call(
        _k6_body,
        grid=(4,),
        in_specs=[
            pl.BlockSpec((NC, NPAD // 4, 128), lambda i: (0, i, 0)),
            pl.BlockSpec((128,), lambda i: (0,)),
        ],
        out_specs=pl.BlockSpec((NPAD // 4, 128), lambda i: (i, 0)),
        out_shape=jax.ShapeDtypeStruct((NPAD, 128), f32),
    )(o_p, bias2)

    return out[:NNODES]
